# baseline (speedup 1.0000x reference)
import functools

import jax
import jax.numpy as jnp
from jax.experimental import pallas as pl

_TOPK = 64

_T_TILE_ENC = 128
_L_TILE_ENC = 1536
_T_TILE_DEC = 2048
_L_TILE_DEC = 1536


def _topk_mask(pre, k, nbits2):
    int_min = jnp.int32(-2147483648)
    rows = pre.shape[0]
    cols = pre.shape[1]
    su = jax.lax.bitcast_convert_type(pre, jnp.int32)
    su = jnp.where(su < 0, su ^ jnp.int32(0x7FFFFFFF), su)
    pre_bf = pre.astype(jnp.bfloat16)

    def _count_bf(tbf):
        m = (pre_bf >= tbf).astype(jnp.bfloat16)
        s1 = jnp.sum(m.reshape(rows, cols // 128, 128), axis=1)
        return jnp.sum(s1.astype(jnp.float32), axis=1, keepdims=True)

    def _bits16_to_bf(bits):
        s = bits ^ jnp.int32(0x8000)
        b = jnp.where((s & 0x8000) != 0, s ^ jnp.int32(0x7FFF), s)
        return jax.lax.bitcast_convert_type(b.astype(jnp.int16), jnp.bfloat16)

    def body1(i, cur):
        cand = cur | jnp.left_shift(jnp.int32(1), 15 - i)
        cnt = _count_bf(_bits16_to_bf(cand))
        return jnp.where(cnt >= k, cand, cur)

    t16 = jax.lax.fori_loop(
        0, 16, body1, jnp.zeros((rows, 1), jnp.int32)
    )
    tv = _bits16_to_bf(t16).astype(jnp.float32)
    pv = _bits16_to_bf(jnp.maximum(t16 - 1, 0)).astype(jnp.float32)
    nv = _bits16_to_bf(jnp.minimum(t16 + 1, 0xFFFF)).astype(jnp.float32)

    def _su_of(v):
        b = jax.lax.bitcast_convert_type(v, jnp.int32)
        return jnp.where(b < 0, b ^ jnp.int32(0x7FFFFFFF), b)

    lo = _su_of(0.5 * (pv + tv)) - 2
    hi = _su_of(0.5 * (tv + nv)) + 2

    def body2(i, carry):
        lo, hi = carry
        mid = lo + jax.lax.shift_right_logical(hi - lo + 1, 1)
        cnt = jnp.sum((su >= mid).astype(jnp.int32), axis=1, keepdims=True)
        ok = cnt >= k
        return (jnp.where(ok, mid, lo), jnp.where(ok, hi, mid - 1))

    lo, hi = jax.lax.fori_loop(0, nbits2, body2, (lo, hi))
    return jnp.where(su >= lo, jnp.maximum(pre, 0.0), 0.0)


def _enc_body(x_ref, we_ref, be_ref, bd_ref, pre_ref, sp_ref, *, k, l_tile, n_l):
    l = pl.program_id(1)
    xc = x_ref[...] - bd_ref[...]
    acc = jnp.dot(xc, we_ref[...], preferred_element_type=jnp.float32)
    pre_ref[:, pl.ds(l * l_tile, l_tile)] = acc + be_ref[...]

    @pl.when(l == n_l - 1)
    def _():
        sp_ref[...] = _topk_mask(pre_ref[...], k, 13)


def _dec_body(sp_ref, wd_ref, bd_ref, out_ref):
    l = pl.program_id(1)

    @pl.when(l == 0)
    def _():
        out_ref[...] = jnp.broadcast_to(bd_ref[...], out_ref.shape)

    out_ref[...] += jnp.dot(
        sp_ref[...], wd_ref[...], preferred_element_type=jnp.float32
    )


def kernel(x, W_enc, b_enc, W_dec, b_dec):
    T, D = x.shape
    L = W_enc.shape[1]

    t_tile = min(_T_TILE_ENC, T)
    l_tile = min(_L_TILE_ENC, L)
    n_t, n_l = T // t_tile, L // l_tile

    pre, sparse = pl.pallas_call(
        functools.partial(_enc_body, k=_TOPK, l_tile=l_tile, n_l=n_l),
        grid=(n_t, n_l),
        in_specs=[
            pl.BlockSpec((t_tile, D), lambda t, l: (t, 0)),
            pl.BlockSpec((D, l_tile), lambda t, l: (0, l)),
            pl.BlockSpec((1, l_tile), lambda t, l: (0, l)),
            pl.BlockSpec((1, D), lambda t, l: (0, 0)),
        ],
        out_specs=[
            pl.BlockSpec((t_tile, L), lambda t, l: (t, 0)),
            pl.BlockSpec((t_tile, L), lambda t, l: (t, 0)),
        ],
        out_shape=[jax.ShapeDtypeStruct((T, L), jnp.float32)] * 2,
    )(x, W_enc, b_enc.reshape(1, L), b_dec.reshape(1, D))

    td_tile = min(_T_TILE_DEC, T)
    ld_tile = min(_L_TILE_DEC, L)
    recon = pl.pallas_call(
        _dec_body,
        grid=(T // td_tile, L // ld_tile),
        in_specs=[
            pl.BlockSpec((td_tile, ld_tile), lambda t, l: (t, l)),
            pl.BlockSpec((ld_tile, D), lambda t, l: (l, 0)),
            pl.BlockSpec((1, D), lambda t, l: (0, 0)),
        ],
        out_specs=pl.BlockSpec((td_tile, D), lambda t, l: (t, 0)),
        out_shape=jax.ShapeDtypeStruct((T, D), jnp.float32),
    )(sparse, W_dec, b_dec.reshape(1, D))

    return (recon, sparse, pre)


# speedup vs baseline: 2.8728x; 2.8728x over previous
import functools

import jax
import jax.numpy as jnp
from jax.experimental import pallas as pl

_TOPK = 64


def _row_threshold(pre, k, nbits):
    int_min = jnp.int32(-2147483648)
    su = jax.lax.bitcast_convert_type(pre, jnp.int32)
    su = jnp.where(su < 0, su ^ jnp.int32(0x7FFFFFFF), su)

    def body(i, cur):
        cand = cur | jnp.left_shift(jnp.int32(1), 31 - i)
        thr = cand ^ int_min
        cnt = jnp.sum((su >= thr).astype(jnp.int32), axis=1, keepdims=True)
        return jnp.where(cnt >= k, cand, cur)

    cur = jax.lax.fori_loop(
        0, nbits, body, jnp.zeros((pre.shape[0], 1), jnp.int32)
    )
    lo = cur ^ int_min
    lo_b = jnp.where(lo < 0, lo ^ jnp.int32(0x7FFFFFFF), lo)
    return jax.lax.bitcast_convert_type(lo_b, jnp.float32)


def _enc_body(x_ref, we_ref, be_ref, bd_ref, pre_ref):
    xc = x_ref[...] - bd_ref[...]
    pre_ref[...] = (
        jnp.dot(xc, we_ref[...], preferred_element_type=jnp.float32)
        + be_ref[...]
    )


def _tau_body(pre_ref, tau_ref, *, k):
    tau_ref[...] = _row_threshold(pre_ref[...], k, 28)


def _maskdec_body(pre_ref, tau_ref, wd_ref, bd_ref, sp_ref, out_ref):
    l = pl.program_id(1)
    p = pre_ref[...]
    sp = jnp.where(p >= tau_ref[...], jnp.maximum(p, 0.0), 0.0)
    sp_ref[...] = sp

    @pl.when(l == 0)
    def _():
        out_ref[...] = jnp.broadcast_to(bd_ref[...], out_ref.shape)

    out_ref[...] += jnp.dot(sp, wd_ref[...], preferred_element_type=jnp.float32)


def kernel(x, W_enc, b_enc, W_dec, b_dec):
    T, D = x.shape
    L = W_enc.shape[1]

    te, le = min(512, T), min(3072, L)
    pre = pl.pallas_call(
        _enc_body,
        grid=(L // le, T // te),
        in_specs=[
            pl.BlockSpec((te, D), lambda l, t: (t, 0)),
            pl.BlockSpec((D, le), lambda l, t: (0, l)),
            pl.BlockSpec((1, le), lambda l, t: (0, l)),
            pl.BlockSpec((1, D), lambda l, t: (0, 0)),
        ],
        out_specs=pl.BlockSpec((te, le), lambda l, t: (t, l)),
        out_shape=jax.ShapeDtypeStruct((T, L), jnp.float32),
    )(x, W_enc, b_enc.reshape(1, L), b_dec.reshape(1, D))

    tt = min(256, T)
    tau = pl.pallas_call(
        functools.partial(_tau_body, k=_TOPK),
        grid=(T // tt,),
        in_specs=[pl.BlockSpec((tt, L), lambda t: (t, 0))],
        out_specs=pl.BlockSpec((tt, 1), lambda t: (t, 0)),
        out_shape=jax.ShapeDtypeStruct((T, 1), jnp.float32),
    )(pre)

    td, ld = min(1024, T), min(1536, L)
    sparse, recon = pl.pallas_call(
        _maskdec_body,
        grid=(T // td, L // ld),
        in_specs=[
            pl.BlockSpec((td, ld), lambda t, l: (t, l)),
            pl.BlockSpec((td, 1), lambda t, l: (t, 0)),
            pl.BlockSpec((ld, D), lambda t, l: (l, 0)),
            pl.BlockSpec((1, D), lambda t, l: (0, 0)),
        ],
        out_specs=[
            pl.BlockSpec((td, ld), lambda t, l: (t, l)),
            pl.BlockSpec((td, D), lambda t, l: (t, 0)),
        ],
        out_shape=[
            jax.ShapeDtypeStruct((T, L), jnp.float32),
            jax.ShapeDtypeStruct((T, D), jnp.float32),
        ],
    )(pre, tau, W_dec, b_dec.reshape(1, D))

    return (recon, sparse, pre)


# speedup vs baseline: 2.9669x; 1.0328x over previous
import functools

import jax
import jax.numpy as jnp
from jax.experimental import pallas as pl

_TOPK = 64


def _row_threshold(pre, k, nbits):
    int_min = jnp.int32(-2147483648)
    su = jax.lax.bitcast_convert_type(pre, jnp.int32)
    su = jnp.where(su < 0, su ^ jnp.int32(0x7FFFFFFF), su)

    def body(i, cur):
        cand = cur | jnp.left_shift(jnp.int32(1), 31 - i)
        thr = cand ^ int_min
        cnt = jnp.sum((su >= thr).astype(jnp.int32), axis=1, keepdims=True)
        return jnp.where(cnt >= k, cand, cur)

    cur = jax.lax.fori_loop(
        0, nbits, body, jnp.zeros((pre.shape[0], 1), jnp.int32)
    )
    lo = cur ^ int_min
    lo_b = jnp.where(lo < 0, lo ^ jnp.int32(0x7FFFFFFF), lo)
    return jax.lax.bitcast_convert_type(lo_b, jnp.float32)


def _enc_body(x_ref, we_ref, be_ref, bd_ref, pre_ref):
    xc = x_ref[...] - bd_ref[...]
    pre_ref[...] = (
        jnp.dot(xc, we_ref[...], preferred_element_type=jnp.float32)
        + be_ref[...]
    )


def _tau_body(pre_ref, tau_ref, *, k):
    tau_ref[...] = _row_threshold(pre_ref[...], k, 28)


def _maskdec_body(pre_ref, tau_ref, wd_ref, bd_ref, sp_ref, out_ref):
    l = pl.program_id(1)
    p = pre_ref[...]
    sp = jnp.where(p >= tau_ref[...], jnp.maximum(p, 0.0), 0.0)
    sp_ref[...] = sp

    @pl.when(l == 0)
    def _():
        out_ref[...] = jnp.broadcast_to(bd_ref[...], out_ref.shape)

    out_ref[...] += jnp.dot(sp, wd_ref[...], preferred_element_type=jnp.float32)


def kernel(x, W_enc, b_enc, W_dec, b_dec):
    T, D = x.shape
    L = W_enc.shape[1]

    te, le = min(1024, T), min(3072, L)
    pre = pl.pallas_call(
        _enc_body,
        grid=(L // le, T // te),
        in_specs=[
            pl.BlockSpec((te, D), lambda l, t: (t, 0)),
            pl.BlockSpec((D, le), lambda l, t: (0, l)),
            pl.BlockSpec((1, le), lambda l, t: (0, l)),
            pl.BlockSpec((1, D), lambda l, t: (0, 0)),
        ],
        out_specs=pl.BlockSpec((te, le), lambda l, t: (t, l)),
        out_shape=jax.ShapeDtypeStruct((T, L), jnp.float32),
    )(x, W_enc, b_enc.reshape(1, L), b_dec.reshape(1, D))

    tt = min(256, T)
    tau = pl.pallas_call(
        functools.partial(_tau_body, k=_TOPK),
        grid=(T // tt,),
        in_specs=[pl.BlockSpec((tt, L), lambda t: (t, 0))],
        out_specs=pl.BlockSpec((tt, 1), lambda t: (t, 0)),
        out_shape=jax.ShapeDtypeStruct((T, 1), jnp.float32),
    )(pre)

    td, ld = min(2048, T), min(768, L)
    sparse, recon = pl.pallas_call(
        _maskdec_body,
        grid=(T // td, L // ld),
        in_specs=[
            pl.BlockSpec((td, ld), lambda t, l: (t, l)),
            pl.BlockSpec((td, 1), lambda t, l: (t, 0)),
            pl.BlockSpec((ld, D), lambda t, l: (l, 0)),
            pl.BlockSpec((1, D), lambda t, l: (0, 0)),
        ],
        out_specs=[
            pl.BlockSpec((td, ld), lambda t, l: (t, l)),
            pl.BlockSpec((td, D), lambda t, l: (t, 0)),
        ],
        out_shape=[
            jax.ShapeDtypeStruct((T, L), jnp.float32),
            jax.ShapeDtypeStruct((T, D), jnp.float32),
        ],
    )(pre, tau, W_dec, b_dec.reshape(1, D))

    return (recon, sparse, pre)


# speedup vs baseline: 2.9672x; 1.0001x over previous
import functools

import jax
import jax.numpy as jnp
from jax.experimental import pallas as pl

_TOPK = 64


def _row_threshold(pre, k, nbits):
    int_min = jnp.int32(-2147483648)
    su = jax.lax.bitcast_convert_type(pre, jnp.int32)
    su = jnp.where(su < 0, su ^ jnp.int32(0x7FFFFFFF), su)

    def body(i, cur):
        cand = cur | jnp.left_shift(jnp.int32(1), 31 - i)
        thr = cand ^ int_min
        cnt = jnp.sum((su >= thr).astype(jnp.int32), axis=1, keepdims=True)
        return jnp.where(cnt >= k, cand, cur)

    cur = jax.lax.fori_loop(
        0, nbits, body, jnp.zeros((pre.shape[0], 1), jnp.int32)
    )
    lo = cur ^ int_min
    lo_b = jnp.where(lo < 0, lo ^ jnp.int32(0x7FFFFFFF), lo)
    return jax.lax.bitcast_convert_type(lo_b, jnp.float32)


def _enc_body(x_ref, we_ref, be_ref, bd_ref, pre_ref):
    xc = (x_ref[...] - bd_ref[...]).astype(jnp.bfloat16)
    pre_ref[...] = (
        jnp.dot(xc, we_ref[...], preferred_element_type=jnp.float32)
        + be_ref[...]
    )


def _tau_body(pre_ref, tau_ref, *, k):
    tau_ref[...] = _row_threshold(pre_ref[...], k, 28)


def _maskdec_body(pre_ref, tau_ref, wd_ref, bd_ref, sp_ref, out_ref):
    l = pl.program_id(1)
    p = pre_ref[...]
    sp = jnp.where(p >= tau_ref[...], jnp.maximum(p, 0.0), 0.0)
    sp_ref[...] = sp

    @pl.when(l == 0)
    def _():
        out_ref[...] = jnp.broadcast_to(bd_ref[...], out_ref.shape)

    out_ref[...] += jnp.dot(sp, wd_ref[...], preferred_element_type=jnp.float32)


def kernel(x, W_enc, b_enc, W_dec, b_dec):
    T, D = x.shape
    L = W_enc.shape[1]

    te, le = min(1024, T), min(3072, L)
    pre = pl.pallas_call(
        _enc_body,
        grid=(L // le, T // te),
        in_specs=[
            pl.BlockSpec((te, D), lambda l, t: (t, 0)),
            pl.BlockSpec((D, le), lambda l, t: (0, l)),
            pl.BlockSpec((1, le), lambda l, t: (0, l)),
            pl.BlockSpec((1, D), lambda l, t: (0, 0)),
        ],
        out_specs=pl.BlockSpec((te, le), lambda l, t: (t, l)),
        out_shape=jax.ShapeDtypeStruct((T, L), jnp.float32),
    )(x, W_enc, b_enc.reshape(1, L), b_dec.reshape(1, D))

    tt = min(256, T)
    tau = pl.pallas_call(
        functools.partial(_tau_body, k=_TOPK),
        grid=(T // tt,),
        in_specs=[pl.BlockSpec((tt, L), lambda t: (t, 0))],
        out_specs=pl.BlockSpec((tt, 1), lambda t: (t, 0)),
        out_shape=jax.ShapeDtypeStruct((T, 1), jnp.float32),
    )(pre)

    td, ld = min(2048, T), min(768, L)
    sparse, recon = pl.pallas_call(
        _maskdec_body,
        grid=(T // td, L // ld),
        in_specs=[
            pl.BlockSpec((td, ld), lambda t, l: (t, l)),
            pl.BlockSpec((td, 1), lambda t, l: (t, 0)),
            pl.BlockSpec((ld, D), lambda t, l: (l, 0)),
            pl.BlockSpec((1, D), lambda t, l: (0, 0)),
        ],
        out_specs=[
            pl.BlockSpec((td, ld), lambda t, l: (t, l)),
            pl.BlockSpec((td, D), lambda t, l: (t, 0)),
        ],
        out_shape=[
            jax.ShapeDtypeStruct((T, L), jnp.float32),
            jax.ShapeDtypeStruct((T, D), jnp.float32),
        ],
    )(pre, tau, W_dec, b_dec.reshape(1, D))

    return (recon, sparse, pre)
